# baseline (speedup 1.0000x reference)
import functools
import math

import jax
import jax.numpy as jnp
from jax import lax
from jax.experimental import pallas as pl
from jax.experimental.pallas import tpu as pltpu
from jax.experimental.pallas import tpu_sc as plsc

N_NODES = 10000
N_PAD = 10240
E = 320000
INV_SQRT8 = 1.0 / math.sqrt(8.0)

_EB = 6400


def _mix_body(shT_ref, w_ref, o_ref):
    shT = shT_ref[...]
    w = w_ref[...]
    kk = lax.broadcasted_iota(jnp.int32, (8, 128), 0)
    cc = lax.broadcasted_iota(jnp.int32, (8, 128), 1)
    R = ((cc // 8) % 8 == kk).astype(jnp.float32)
    shb = lax.dot_general(
        shT, R, (((0,), (0,)), ((), ())),
        preferred_element_type=jnp.float32)
    p = shb * w
    p8 = p.reshape(_EB // 8, 1024)
    a1 = lax.broadcasted_iota(jnp.int32, (1024, 128), 0) // 128
    a2 = lax.broadcasted_iota(jnp.int32, (1024, 128), 1) // 16
    c3 = lax.broadcasted_iota(jnp.int32, (1024, 128), 0) % 128
    d3 = lax.broadcasted_iota(jnp.int32, (1024, 128), 1) % 16
    S8 = ((a1 == a2) & (c3 // 64 == d3 // 8)
          & (c3 % 8 == d3 % 8)).astype(jnp.float32)
    o_ref[...] = jnp.dot(
        p8.astype(jnp.bfloat16), S8.astype(jnp.bfloat16),
        preferred_element_type=jnp.float32) * INV_SQRT8


def _mix_call(edge_sh, weight):
    eshT = edge_sh.T
    return pl.pallas_call(
        _mix_body,
        grid=(E // _EB,),
        in_specs=[
            pl.BlockSpec((8, _EB), lambda i: (0, i)),
            pl.BlockSpec((_EB, 128), lambda i: (i, 0)),
        ],
        out_specs=pl.BlockSpec((_EB // 8, 128), lambda i: (i, 0)),
        out_shape=jax.ShapeDtypeStruct((E // 8, 128), jnp.float32),
    )(eshT, weight)


_NC = 2
_NS = 16
_NW = _NC * _NS
_EPW = E // _NW
_SC_CHUNK = 128
_SC_FULL = _EPW // _SC_CHUNK
_SC_TAIL = _EPW - _SC_FULL * _SC_CHUNK
_ROWS_PT = N_PAD // _NS

_sc_mesh = plsc.VectorSubcoreMesh(core_axis_name="c", subcore_axis_name="s")
_sc_params = pltpu.CompilerParams(
    needs_layout_passes=False, use_tc_tiling_on_sc=False
)


@functools.partial(
    pl.kernel,
    mesh=_sc_mesh,
    compiler_params=_sc_params,
    out_type=jax.ShapeDtypeStruct((_NC, N_PAD, 16), jnp.float32),
    scratch_types=[
        pltpu.VMEM_SHARED((N_PAD, 16), jnp.float32),
        pltpu.VMEM((_ROWS_PT, 16), jnp.float32),
        pltpu.VMEM((_SC_CHUNK,), jnp.int32),
        pltpu.VMEM((_SC_CHUNK,), jnp.int32),
        pltpu.VMEM((_SC_CHUNK, 16), jnp.float32),
        pltpu.VMEM((_SC_CHUNK, 16), jnp.float32),
        pltpu.SemaphoreType.DMA,
        pltpu.SemaphoreType.DMA,
        pltpu.SemaphoreType.DMA,
        pltpu.SemaphoreType.DMA,
    ],
)
def _scatter_kernel(mix_hbm, ei_hbm, part_hbm, table, zed,
                    idx0, idx1, rows0, rows1, is0, is1, rs0, rs1):
    cid = lax.axis_index("c")
    sid = lax.axis_index("s")
    wid = sid * _NC + cid
    idxb, rowsb = (idx0, idx1), (rows0, rows1)
    isem, rsem = (is0, is1), (rs0, rs1)

    z16 = jnp.zeros((16,), jnp.float32)

    def _z(i, _):
        zed[i, :] = z16
        return 0

    lax.fori_loop(0, _ROWS_PT, _z, 0)
    pltpu.sync_copy(zed, table.at[pl.ds(sid * _ROWS_PT, _ROWS_PT)])
    plsc.subcore_barrier()

    base = wid * _EPW

    def _load(c, b):
        off = pl.multiple_of(base + c * _SC_CHUNK, 8)
        pltpu.async_copy(ei_hbm.at[0, pl.ds(off, _SC_CHUNK)], idxb[b], isem[b])
        pltpu.async_copy(mix_hbm.at[pl.ds(off, _SC_CHUNK)], rowsb[b], rsem[b])

    _load(0, 0)
    _load(1, 1)

    def _body(cc, _):
        for b in range(2):
            c = cc * 2 + b
            pltpu.make_async_copy(
                ei_hbm.at[0, pl.ds(base, _SC_CHUNK)], idxb[b], isem[b]).wait()
            pltpu.make_async_copy(
                mix_hbm.at[pl.ds(base, _SC_CHUNK)], rowsb[b], rsem[b]).wait()
            pltpu.sync_copy(rowsb[b], table.at[idxb[b]], add=True)

            @pl.when(c + 2 < _SC_FULL)
            def _():
                _load(c + 2, b)

        return 0

    lax.fori_loop(0, _SC_FULL // 2, _body, 0)

    if _SC_TAIL:
        toff = base + _SC_FULL * _SC_CHUNK
        pltpu.sync_copy(ei_hbm.at[0, pl.ds(toff, _SC_TAIL)],
                        idx0.at[pl.ds(0, _SC_TAIL)])
        pltpu.sync_copy(mix_hbm.at[pl.ds(toff, _SC_TAIL)],
                        rows0.at[pl.ds(0, _SC_TAIL)])
        pltpu.sync_copy(rows0.at[pl.ds(0, _SC_TAIL)],
                        table.at[idx0.at[pl.ds(0, _SC_TAIL)]], add=True)
    plsc.subcore_barrier()

    pltpu.sync_copy(
        table.at[pl.ds(sid * _ROWS_PT, _ROWS_PT)],
        part_hbm.at[cid, pl.ds(sid * _ROWS_PT, _ROWS_PT)],
    )


_GC = 64
_G_FULL = _EPW // _GC
_G_TAIL = _EPW - _G_FULL * _GC
_STG_PT = 2 * N_PAD // _NS


@functools.partial(
    pl.kernel,
    mesh=_sc_mesh,
    compiler_params=_sc_params,
    out_type=jax.ShapeDtypeStruct((2 * E, 8), jnp.float32),
    scratch_types=[
        pltpu.VMEM_SHARED((2 * N_PAD, 8), jnp.float32),
        pltpu.VMEM((_STG_PT, 8), jnp.float32),
        pltpu.VMEM((_GC,), jnp.int32),
        pltpu.VMEM((_GC,), jnp.int32),
        pltpu.VMEM((_GC,), jnp.int32),
        pltpu.VMEM((_GC,), jnp.int32),
        pltpu.VMEM((2 * _GC,), jnp.int32),
        pltpu.VMEM((2 * _GC,), jnp.int32),
        pltpu.VMEM((2 * _GC, 8), jnp.float32),
        pltpu.VMEM((2 * _GC, 8), jnp.float32),
        pltpu.SemaphoreType.DMA,
        pltpu.SemaphoreType.DMA,
        pltpu.SemaphoreType.DMA,
        pltpu.SemaphoreType.DMA,
        pltpu.SemaphoreType.DMA,
        pltpu.SemaphoreType.DMA,
    ],
)
def _gather_kernel(p8_hbm, ei_hbm, out_hbm, table, stg,
                   ii0, ii1, ij0, ij1, cx0, cx1, rw0, rw1,
                   il0, il1, gs0, gs1, os0, os1):
    cid = lax.axis_index("c")
    sid = lax.axis_index("s")
    wid = sid * _NC + cid
    iib, ijb = (ii0, ii1), (ij0, ij1)
    cxb, rwb = (cx0, cx1), (rw0, rw1)
    ilsem, gsem, osem = (il0, il1), (gs0, gs1), (os0, os1)

    lane = lax.iota(jnp.int32, 16)

    r0 = sid * _STG_PT
    pltpu.sync_copy(p8_hbm.at[0, pl.ds(r0, _STG_PT)], stg)
    pltpu.sync_copy(stg, table.at[pl.ds(r0, _STG_PT)])
    for c in range(_STG_PT // 128):
        pltpu.sync_copy(p8_hbm.at[1, pl.ds(r0 + 128 * c, 128)], rw0)
        for k in range(8):
            cx0[pl.ds(16 * k, 16)] = lane + (r0 + 128 * c + 16 * k)
        pltpu.sync_copy(rw0, table.at[cx0], add=True)
    plsc.subcore_barrier()

    half = lax.shift_right_logical(lane, 1)
    even = (lane & 1) == 0
    base = wid * _EPW

    def _load_idx(c, b):
        eoff = pl.multiple_of(base + c * _GC, 8)
        pltpu.async_copy(ei_hbm.at[0, pl.ds(eoff, _GC)], iib[b], ilsem[b])
        pltpu.async_copy(ei_hbm.at[1, pl.ds(eoff, _GC)], ijb[b], ilsem[b])

    def _build_cidx(b, n_edges):
        for g in range(n_edges // 8):
            idxg = half + (8 * g)
            va = plsc.load_gather(iib[b], [idxg])
            vb = plsc.load_gather(ijb[b], [idxg])
            cxb[b][pl.ds(16 * g, 16)] = jnp.where(even, 2 * va, 2 * vb + 1)

    _load_idx(0, 0)
    _load_idx(1, 1)

    def _body(cc, _):
        for b in range(2):
            c = cc * 2 + b
            pltpu.make_async_copy(
                ei_hbm.at[0, pl.ds(base, _GC)], iib[b], ilsem[b]).wait()
            pltpu.make_async_copy(
                ei_hbm.at[1, pl.ds(base, _GC)], ijb[b], ilsem[b]).wait()
            _build_cidx(b, _GC)

            @pl.when(c >= 2)
            def _():
                pltpu.make_async_copy(
                    rwb[b], out_hbm.at[pl.ds(0, 2 * _GC)], osem[b]).wait()

            pltpu.async_copy(table.at[cxb[b]], rwb[b], gsem[b])

            @pl.when(c + 2 < _G_FULL)
            def _():
                _load_idx(c + 2, b)

            eoff = pl.multiple_of(base + c * _GC, 8)
            pltpu.make_async_copy(table.at[cxb[b]], rwb[b], gsem[b]).wait()
            pltpu.async_copy(rwb[b], out_hbm.at[pl.ds(2 * eoff, 2 * _GC)],
                             osem[b])
        return 0

    lax.fori_loop(0, _G_FULL // 2, _body, 0)
    for b in range(2):
        pltpu.make_async_copy(
            rwb[b], out_hbm.at[pl.ds(0, 2 * _GC)], osem[b]).wait()

    if _G_TAIL:
        eoff = base + _G_FULL * _GC
        pltpu.sync_copy(ei_hbm.at[0, pl.ds(eoff, _G_TAIL)],
                        ii0.at[pl.ds(0, _G_TAIL)])
        pltpu.sync_copy(ei_hbm.at[1, pl.ds(eoff, _G_TAIL)],
                        ij0.at[pl.ds(0, _G_TAIL)])
        _build_cidx(0, _G_TAIL)
        pltpu.async_copy(
            table.at[cx0.at[pl.ds(0, 2 * _G_TAIL)]],
            rw0.at[pl.ds(0, 2 * _G_TAIL)], gs0).wait()
        pltpu.sync_copy(rw0.at[pl.ds(0, 2 * _G_TAIL)],
                        out_hbm.at[pl.ds(2 * eoff, 2 * _G_TAIL)])


_UB = 6400


def _unpack_body(v_ref, o_ref):
    v = v_ref[...]
    r0 = lax.broadcasted_iota(jnp.int32, (128, 1024), 0)
    c0 = lax.broadcasted_iota(jnp.int32, (128, 1024), 1)
    Q = ((r0 // 16 == c0 // 128) & (c0 % 128 == r0 % 16)).astype(jnp.float32)
    u = jnp.dot(v.astype(jnp.bfloat16), Q.astype(jnp.bfloat16),
                preferred_element_type=jnp.float32)
    m = u.reshape(_UB, 128)[:, :16]
    o_ref[...] = m.T


def _unpack_call(out8):
    v = out8.reshape(E // 8, 128)
    outT = pl.pallas_call(
        _unpack_body,
        grid=(E // _UB,),
        in_specs=[pl.BlockSpec((_UB // 8, 128), lambda i: (i, 0))],
        out_specs=pl.BlockSpec((16, _UB), lambda i: (0, i)),
        out_shape=jax.ShapeDtypeStruct((16, E), jnp.float32),
    )(v)
    return outT.T


def kernel(edge_sh, weight, edge_index):
    mix8 = _mix_call(edge_sh, weight)
    mix = mix8.reshape(E, 16)
    partials = _scatter_kernel(mix, edge_index)
    p8 = partials.reshape(2, 2 * N_PAD, 8)
    out8 = _gather_kernel(p8, edge_index)
    return _unpack_call(out8)


# speedup vs baseline: 1.0259x; 1.0259x over previous
import functools
import math

import jax
import jax.numpy as jnp
from jax import lax
from jax.experimental import pallas as pl
from jax.experimental.pallas import tpu as pltpu
from jax.experimental.pallas import tpu_sc as plsc

N_NODES = 10000
N_PAD = 10240
E = 320000
INV_SQRT8 = 1.0 / math.sqrt(8.0)

_EB = 6400


def _mix_body(shT_ref, w_ref, o_ref):
    shT = shT_ref[...]
    w = w_ref[...]
    kk = lax.broadcasted_iota(jnp.int32, (8, 128), 0)
    cc = lax.broadcasted_iota(jnp.int32, (8, 128), 1)
    R = ((cc // 8) % 8 == kk).astype(jnp.float32)
    shb = lax.dot_general(
        shT, R, (((0,), (0,)), ((), ())),
        preferred_element_type=jnp.float32)
    p = shb * w
    p8 = p.reshape(_EB // 8, 1024)
    a1 = lax.broadcasted_iota(jnp.int32, (1024, 128), 0) // 128
    a2 = lax.broadcasted_iota(jnp.int32, (1024, 128), 1) // 16
    c3 = lax.broadcasted_iota(jnp.int32, (1024, 128), 0) % 128
    d3 = lax.broadcasted_iota(jnp.int32, (1024, 128), 1) % 16
    S8 = ((a1 == a2) & (c3 // 64 == d3 // 8)
          & (c3 % 8 == d3 % 8)).astype(jnp.float32)
    o_ref[...] = jnp.dot(
        p8.astype(jnp.bfloat16), S8.astype(jnp.bfloat16),
        preferred_element_type=jnp.float32) * INV_SQRT8


def _mix_call(edge_sh, weight):
    eshT = edge_sh.T
    return pl.pallas_call(
        _mix_body,
        grid=(E // _EB,),
        in_specs=[
            pl.BlockSpec((8, _EB), lambda i: (0, i)),
            pl.BlockSpec((_EB, 128), lambda i: (i, 0)),
        ],
        out_specs=pl.BlockSpec((_EB // 8, 128), lambda i: (i, 0)),
        out_shape=jax.ShapeDtypeStruct((E // 8, 128), jnp.float32),
    )(eshT, weight)


def _add_body(p_ref, o_ref):
    o_ref[...] = p_ref[0] + p_ref[1]


def _add_call(partials):
    p8 = partials.reshape(2, N_PAD * 16 // 128, 128)
    return pl.pallas_call(
        _add_body,
        out_shape=jax.ShapeDtypeStruct((N_PAD * 16 // 128, 128), jnp.float32),
    )(p8)


_NC = 2
_NS = 16
_NW = _NC * _NS
_EPW = E // _NW
_SC_CHUNK = 128
_SC_FULL = _EPW // _SC_CHUNK
_SC_TAIL = _EPW - _SC_FULL * _SC_CHUNK
_ROWS_PT = N_PAD // _NS

_sc_mesh = plsc.VectorSubcoreMesh(core_axis_name="c", subcore_axis_name="s")
_sc_params = pltpu.CompilerParams(
    needs_layout_passes=False, use_tc_tiling_on_sc=False
)


@functools.partial(
    pl.kernel,
    mesh=_sc_mesh,
    compiler_params=_sc_params,
    out_type=jax.ShapeDtypeStruct((_NC, N_PAD, 16), jnp.float32),
    scratch_types=[
        pltpu.VMEM_SHARED((N_PAD, 16), jnp.float32),
        pltpu.VMEM((_ROWS_PT, 16), jnp.float32),
        pltpu.VMEM((_SC_CHUNK,), jnp.int32),
        pltpu.VMEM((_SC_CHUNK,), jnp.int32),
        pltpu.VMEM((_SC_CHUNK, 16), jnp.float32),
        pltpu.VMEM((_SC_CHUNK, 16), jnp.float32),
        pltpu.SemaphoreType.DMA,
        pltpu.SemaphoreType.DMA,
        pltpu.SemaphoreType.DMA,
        pltpu.SemaphoreType.DMA,
    ],
)
def _scatter_kernel(mix_hbm, ei_hbm, part_hbm, table, zed,
                    idx0, idx1, rows0, rows1, is0, is1, rs0, rs1):
    cid = lax.axis_index("c")
    sid = lax.axis_index("s")
    wid = sid * _NC + cid
    idxb, rowsb = (idx0, idx1), (rows0, rows1)
    isem, rsem = (is0, is1), (rs0, rs1)

    z16 = jnp.zeros((16,), jnp.float32)

    def _z(i, _):
        zed[i, :] = z16
        return 0

    lax.fori_loop(0, _ROWS_PT, _z, 0)
    pltpu.sync_copy(zed, table.at[pl.ds(sid * _ROWS_PT, _ROWS_PT)])
    plsc.subcore_barrier()

    base = wid * _EPW

    def _load(c, b):
        off = pl.multiple_of(base + c * _SC_CHUNK, 8)
        pltpu.async_copy(ei_hbm.at[0, pl.ds(off, _SC_CHUNK)], idxb[b], isem[b])
        pltpu.async_copy(mix_hbm.at[pl.ds(off, _SC_CHUNK)], rowsb[b], rsem[b])

    _load(0, 0)
    _load(1, 1)

    def _body(cc, _):
        for b in range(2):
            c = cc * 2 + b
            pltpu.make_async_copy(
                ei_hbm.at[0, pl.ds(base, _SC_CHUNK)], idxb[b], isem[b]).wait()
            pltpu.make_async_copy(
                mix_hbm.at[pl.ds(base, _SC_CHUNK)], rowsb[b], rsem[b]).wait()
            pltpu.sync_copy(rowsb[b], table.at[idxb[b]], add=True)

            @pl.when(c + 2 < _SC_FULL)
            def _():
                _load(c + 2, b)

        return 0

    lax.fori_loop(0, _SC_FULL // 2, _body, 0)

    if _SC_TAIL:
        toff = base + _SC_FULL * _SC_CHUNK
        pltpu.sync_copy(ei_hbm.at[0, pl.ds(toff, _SC_TAIL)],
                        idx0.at[pl.ds(0, _SC_TAIL)])
        pltpu.sync_copy(mix_hbm.at[pl.ds(toff, _SC_TAIL)],
                        rows0.at[pl.ds(0, _SC_TAIL)])
        pltpu.sync_copy(rows0.at[pl.ds(0, _SC_TAIL)],
                        table.at[idx0.at[pl.ds(0, _SC_TAIL)]], add=True)
    plsc.subcore_barrier()

    pltpu.sync_copy(
        table.at[pl.ds(sid * _ROWS_PT, _ROWS_PT)],
        part_hbm.at[cid, pl.ds(sid * _ROWS_PT, _ROWS_PT)],
    )


_GC = 64
_G_FULL = _EPW // _GC
_G_TAIL = _EPW - _G_FULL * _GC
_STG_PT = 2 * N_PAD // _NS


@functools.partial(
    pl.kernel,
    mesh=_sc_mesh,
    compiler_params=_sc_params,
    out_type=jax.ShapeDtypeStruct((2 * E, 8), jnp.float32),
    scratch_types=[
        pltpu.VMEM_SHARED((2 * N_PAD, 8), jnp.float32),
        pltpu.VMEM((_STG_PT, 8), jnp.float32),
        pltpu.VMEM((_GC,), jnp.int32),
        pltpu.VMEM((_GC,), jnp.int32),
        pltpu.VMEM((_GC,), jnp.int32),
        pltpu.VMEM((_GC,), jnp.int32),
        pltpu.VMEM((2 * _GC,), jnp.int32),
        pltpu.VMEM((2 * _GC,), jnp.int32),
        pltpu.VMEM((2 * _GC, 8), jnp.float32),
        pltpu.VMEM((2 * _GC, 8), jnp.float32),
        pltpu.SemaphoreType.DMA,
        pltpu.SemaphoreType.DMA,
        pltpu.SemaphoreType.DMA,
        pltpu.SemaphoreType.DMA,
        pltpu.SemaphoreType.DMA,
        pltpu.SemaphoreType.DMA,
    ],
)
def _gather_kernel(m8_hbm, ei_hbm, out_hbm, table, stg,
                   ii0, ii1, ij0, ij1, cx0, cx1, rw0, rw1,
                   il0, il1, gs0, gs1, os0, os1):
    cid = lax.axis_index("c")
    sid = lax.axis_index("s")
    wid = sid * _NC + cid
    iib, ijb = (ii0, ii1), (ij0, ij1)
    cxb, rwb = (cx0, cx1), (rw0, rw1)
    ilsem, gsem, osem = (il0, il1), (gs0, gs1), (os0, os1)

    lane = lax.iota(jnp.int32, 16)

    r0 = sid * _STG_PT
    pltpu.sync_copy(m8_hbm.at[pl.ds(r0, _STG_PT)], stg)
    pltpu.sync_copy(stg, table.at[pl.ds(r0, _STG_PT)])
    plsc.subcore_barrier()

    half = lax.shift_right_logical(lane, 1)
    even = (lane & 1) == 0
    base = wid * _EPW

    def _load_idx(c, b):
        eoff = pl.multiple_of(base + c * _GC, 8)
        pltpu.async_copy(ei_hbm.at[0, pl.ds(eoff, _GC)], iib[b], ilsem[b])
        pltpu.async_copy(ei_hbm.at[1, pl.ds(eoff, _GC)], ijb[b], ilsem[b])

    def _build_cidx(b, n_edges):
        for g in range(n_edges // 8):
            idxg = half + (8 * g)
            va = plsc.load_gather(iib[b], [idxg])
            vb = plsc.load_gather(ijb[b], [idxg])
            cxb[b][pl.ds(16 * g, 16)] = jnp.where(even, 2 * va, 2 * vb + 1)

    _load_idx(0, 0)
    _load_idx(1, 1)

    def _body(cc, _):
        for b in range(2):
            c = cc * 2 + b
            pltpu.make_async_copy(
                ei_hbm.at[0, pl.ds(base, _GC)], iib[b], ilsem[b]).wait()
            pltpu.make_async_copy(
                ei_hbm.at[1, pl.ds(base, _GC)], ijb[b], ilsem[b]).wait()
            _build_cidx(b, _GC)

            @pl.when(c >= 2)
            def _():
                pltpu.make_async_copy(
                    rwb[b], out_hbm.at[pl.ds(0, 2 * _GC)], osem[b]).wait()

            pltpu.async_copy(table.at[cxb[b]], rwb[b], gsem[b])

            @pl.when(c + 2 < _G_FULL)
            def _():
                _load_idx(c + 2, b)

            eoff = pl.multiple_of(base + c * _GC, 8)
            pltpu.make_async_copy(table.at[cxb[b]], rwb[b], gsem[b]).wait()
            pltpu.async_copy(rwb[b], out_hbm.at[pl.ds(2 * eoff, 2 * _GC)],
                             osem[b])
        return 0

    lax.fori_loop(0, _G_FULL // 2, _body, 0)
    for b in range(2):
        pltpu.make_async_copy(
            rwb[b], out_hbm.at[pl.ds(0, 2 * _GC)], osem[b]).wait()

    if _G_TAIL:
        eoff = base + _G_FULL * _GC
        pltpu.sync_copy(ei_hbm.at[0, pl.ds(eoff, _G_TAIL)],
                        ii0.at[pl.ds(0, _G_TAIL)])
        pltpu.sync_copy(ei_hbm.at[1, pl.ds(eoff, _G_TAIL)],
                        ij0.at[pl.ds(0, _G_TAIL)])
        _build_cidx(0, _G_TAIL)
        pltpu.async_copy(
            table.at[cx0.at[pl.ds(0, 2 * _G_TAIL)]],
            rw0.at[pl.ds(0, 2 * _G_TAIL)], gs0).wait()
        pltpu.sync_copy(rw0.at[pl.ds(0, 2 * _G_TAIL)],
                        out_hbm.at[pl.ds(2 * eoff, 2 * _G_TAIL)])


_UB = 6400


def _unpack_body(v_ref, o_ref):
    v = v_ref[...]
    r0 = lax.broadcasted_iota(jnp.int32, (128, 1024), 0)
    c0 = lax.broadcasted_iota(jnp.int32, (128, 1024), 1)
    Q = ((r0 // 16 == c0 // 128) & (c0 % 128 == r0 % 16)).astype(jnp.float32)
    u = jnp.dot(v.astype(jnp.bfloat16), Q.astype(jnp.bfloat16),
                preferred_element_type=jnp.float32)
    m = u.reshape(_UB, 128)[:, :16]
    o_ref[...] = m.T


def _unpack_call(out8):
    v = out8.reshape(E // 8, 128)
    outT = pl.pallas_call(
        _unpack_body,
        grid=(E // _UB,),
        in_specs=[pl.BlockSpec((_UB // 8, 128), lambda i: (i, 0))],
        out_specs=pl.BlockSpec((16, _UB), lambda i: (0, i)),
        out_shape=jax.ShapeDtypeStruct((16, E), jnp.float32),
    )(v)
    return outT.T


def kernel(edge_sh, weight, edge_index):
    mix8 = _mix_call(edge_sh, weight)
    mix = mix8.reshape(E, 16)
    partials = _scatter_kernel(mix, edge_index)
    merged = _add_call(partials)
    m8 = merged.reshape(2 * N_PAD, 8)
    out8 = _gather_kernel(m8, edge_index)
    return _unpack_call(out8)


# speedup vs baseline: 1.0951x; 1.0675x over previous
import functools
import math

import jax
import jax.numpy as jnp
from jax import lax
from jax.experimental import pallas as pl
from jax.experimental.pallas import tpu as pltpu
from jax.experimental.pallas import tpu_sc as plsc

N_NODES = 10000
N_PAD = 10240
E = 320000
INV_SQRT8 = 1.0 / math.sqrt(8.0)

_EB = 6400


def _mix_body(shT_ref, w_ref, o_ref):
    shT = shT_ref[...]
    w = w_ref[...]
    kk = lax.broadcasted_iota(jnp.int32, (8, 128), 0)
    cc = lax.broadcasted_iota(jnp.int32, (8, 128), 1)
    R = ((cc // 8) % 8 == kk).astype(jnp.float32)
    shb = lax.dot_general(
        shT, R, (((0,), (0,)), ((), ())),
        preferred_element_type=jnp.float32)
    p = shb * w
    p8 = p.reshape(_EB // 8, 1024)
    a1 = lax.broadcasted_iota(jnp.int32, (1024, 128), 0) // 128
    a2 = lax.broadcasted_iota(jnp.int32, (1024, 128), 1) // 16
    c3 = lax.broadcasted_iota(jnp.int32, (1024, 128), 0) % 128
    d3 = lax.broadcasted_iota(jnp.int32, (1024, 128), 1) % 16
    S8 = ((a1 == a2) & (c3 // 64 == d3 // 8)
          & (c3 % 8 == d3 % 8)).astype(jnp.float32)
    o_ref[...] = jnp.dot(
        p8.astype(jnp.bfloat16), S8.astype(jnp.bfloat16),
        preferred_element_type=jnp.float32) * INV_SQRT8


def _mix_call(edge_sh, weight):
    eshT = edge_sh.T
    return pl.pallas_call(
        _mix_body,
        grid=(E // _EB,),
        in_specs=[
            pl.BlockSpec((8, _EB), lambda i: (0, i)),
            pl.BlockSpec((_EB, 128), lambda i: (i, 0)),
        ],
        out_specs=pl.BlockSpec((_EB // 8, 128), lambda i: (i, 0)),
        out_shape=jax.ShapeDtypeStruct((E // 8, 128), jnp.float32),
    )(eshT, weight)


def _add_body(p_ref, o_ref):
    o_ref[...] = p_ref[0] + p_ref[1]


def _add_call(partials):
    p8 = partials.reshape(2, N_PAD * 16 // 128, 128)
    return pl.pallas_call(
        _add_body,
        out_shape=jax.ShapeDtypeStruct((N_PAD * 16 // 128, 128), jnp.float32),
    )(p8)


_NC = 2
_NS = 16
_NW = _NC * _NS
_EPW = E // _NW
_SC_CHUNK = 128
_SC_FULL = _EPW // _SC_CHUNK
_SC_TAIL = _EPW - _SC_FULL * _SC_CHUNK
_ROWS_PT = N_PAD // _NS

_sc_mesh = plsc.VectorSubcoreMesh(core_axis_name="c", subcore_axis_name="s")
_sc_params = pltpu.CompilerParams(
    needs_layout_passes=False, use_tc_tiling_on_sc=False
)


@functools.partial(
    pl.kernel,
    mesh=_sc_mesh,
    compiler_params=_sc_params,
    out_type=jax.ShapeDtypeStruct((_NC, N_PAD, 16), jnp.float32),
    scratch_types=[
        pltpu.VMEM_SHARED((N_PAD, 16), jnp.float32),
        pltpu.VMEM((_ROWS_PT, 16), jnp.float32),
        pltpu.VMEM((_SC_CHUNK,), jnp.int32),
        pltpu.VMEM((_SC_CHUNK,), jnp.int32),
        pltpu.VMEM((_SC_CHUNK, 16), jnp.float32),
        pltpu.VMEM((_SC_CHUNK, 16), jnp.float32),
        pltpu.SemaphoreType.DMA,
        pltpu.SemaphoreType.DMA,
        pltpu.SemaphoreType.DMA,
        pltpu.SemaphoreType.DMA,
    ],
)
def _scatter_kernel(mix_hbm, ei_hbm, part_hbm, table, zed,
                    idx0, idx1, rows0, rows1, is0, is1, rs0, rs1):
    cid = lax.axis_index("c")
    sid = lax.axis_index("s")
    wid = sid * _NC + cid
    idxb, rowsb = (idx0, idx1), (rows0, rows1)
    isem, rsem = (is0, is1), (rs0, rs1)

    z16 = jnp.zeros((16,), jnp.float32)

    def _z(i, _):
        zed[i, :] = z16
        return 0

    lax.fori_loop(0, _ROWS_PT, _z, 0)
    pltpu.sync_copy(zed, table.at[pl.ds(sid * _ROWS_PT, _ROWS_PT)])
    plsc.subcore_barrier()

    base = wid * _EPW

    def _load(c, b):
        off = pl.multiple_of(base + c * _SC_CHUNK, 8)
        pltpu.async_copy(ei_hbm.at[0, pl.ds(off, _SC_CHUNK)], idxb[b], isem[b])
        pltpu.async_copy(mix_hbm.at[pl.ds(off, _SC_CHUNK)], rowsb[b], rsem[b])

    _load(0, 0)
    _load(1, 1)

    def _body(cc, _):
        for b in range(2):
            c = cc * 2 + b
            pltpu.make_async_copy(
                ei_hbm.at[0, pl.ds(base, _SC_CHUNK)], idxb[b], isem[b]).wait()
            pltpu.make_async_copy(
                mix_hbm.at[pl.ds(base, _SC_CHUNK)], rowsb[b], rsem[b]).wait()
            pltpu.sync_copy(rowsb[b], table.at[idxb[b]], add=True)

            @pl.when(c + 2 < _SC_FULL)
            def _():
                _load(c + 2, b)

        return 0

    lax.fori_loop(0, _SC_FULL // 2, _body, 0)

    if _SC_TAIL:
        toff = base + _SC_FULL * _SC_CHUNK
        pltpu.sync_copy(ei_hbm.at[0, pl.ds(toff, _SC_TAIL)],
                        idx0.at[pl.ds(0, _SC_TAIL)])
        pltpu.sync_copy(mix_hbm.at[pl.ds(toff, _SC_TAIL)],
                        rows0.at[pl.ds(0, _SC_TAIL)])
        pltpu.sync_copy(rows0.at[pl.ds(0, _SC_TAIL)],
                        table.at[idx0.at[pl.ds(0, _SC_TAIL)]], add=True)
    plsc.subcore_barrier()

    pltpu.sync_copy(
        table.at[pl.ds(sid * _ROWS_PT, _ROWS_PT)],
        part_hbm.at[cid, pl.ds(sid * _ROWS_PT, _ROWS_PT)],
    )


_GC = 64
_EH = E // 2
_EPW_G = _EH // _NW
_G_FULL = _EPW_G // _GC
_G_TAIL = _EPW_G - _G_FULL * _GC
_STG_PT = 2 * N_PAD // _NS


def _make_gather(hpart):
  @functools.partial(
    pl.kernel,
    mesh=_sc_mesh,
    compiler_params=_sc_params,
    out_type=jax.ShapeDtypeStruct((2 * _EH, 8), jnp.float32),
    scratch_types=[
        pltpu.VMEM_SHARED((2 * N_PAD, 8), jnp.float32),
        pltpu.VMEM((_STG_PT, 8), jnp.float32),
        pltpu.VMEM((_GC,), jnp.int32),
        pltpu.VMEM((_GC,), jnp.int32),
        pltpu.VMEM((_GC,), jnp.int32),
        pltpu.VMEM((_GC,), jnp.int32),
        pltpu.VMEM((2 * _GC,), jnp.int32),
        pltpu.VMEM((2 * _GC,), jnp.int32),
        pltpu.VMEM((2 * _GC, 8), jnp.float32),
        pltpu.VMEM((2 * _GC, 8), jnp.float32),
        pltpu.SemaphoreType.DMA,
        pltpu.SemaphoreType.DMA,
        pltpu.SemaphoreType.DMA,
        pltpu.SemaphoreType.DMA,
        pltpu.SemaphoreType.DMA,
        pltpu.SemaphoreType.DMA,
    ],
  )
  def _gather_kernel(m8_hbm, ei_hbm, out_hbm, table, stg,
                     ii0, ii1, ij0, ij1, cx0, cx1, rw0, rw1,
                     il0, il1, gs0, gs1, os0, os1):
      cid = lax.axis_index("c")
      sid = lax.axis_index("s")
      wid = sid * _NC + cid
      iib, ijb = (ii0, ii1), (ij0, ij1)
      cxb, rwb = (cx0, cx1), (rw0, rw1)
      ilsem, gsem, osem = (il0, il1), (gs0, gs1), (os0, os1)

      lane = lax.iota(jnp.int32, 16)

      r0 = sid * _STG_PT
      pltpu.sync_copy(m8_hbm.at[pl.ds(r0, _STG_PT)], stg)
      pltpu.sync_copy(stg, table.at[pl.ds(r0, _STG_PT)])
      plsc.subcore_barrier()

      half = lax.shift_right_logical(lane, 1)
      even = (lane & 1) == 0
      base = hpart * _EH + wid * _EPW_G

      def _load_idx(c, b):
          eoff = pl.multiple_of(base + c * _GC, 8)
          pltpu.async_copy(ei_hbm.at[0, pl.ds(eoff, _GC)], iib[b], ilsem[b])
          pltpu.async_copy(ei_hbm.at[1, pl.ds(eoff, _GC)], ijb[b], ilsem[b])

      def _build_cidx(b, n_edges):
          for g in range(n_edges // 8):
              idxg = half + (8 * g)
              va = plsc.load_gather(iib[b], [idxg])
              vb = plsc.load_gather(ijb[b], [idxg])
              cxb[b][pl.ds(16 * g, 16)] = jnp.where(even, 2 * va, 2 * vb + 1)

      _load_idx(0, 0)
      _load_idx(1, 1)

      def _body(cc, _):
          for b in range(2):
              c = cc * 2 + b
              pltpu.make_async_copy(
                  ei_hbm.at[0, pl.ds(base, _GC)], iib[b], ilsem[b]).wait()
              pltpu.make_async_copy(
                  ei_hbm.at[1, pl.ds(base, _GC)], ijb[b], ilsem[b]).wait()
              _build_cidx(b, _GC)

              @pl.when(c >= 2)
              def _():
                  pltpu.make_async_copy(
                      rwb[b], out_hbm.at[pl.ds(0, 2 * _GC)], osem[b]).wait()

              pltpu.async_copy(table.at[cxb[b]], rwb[b], gsem[b])

              @pl.when(c + 2 < _G_FULL)
              def _():
                  _load_idx(c + 2, b)

              eoff = pl.multiple_of(base + c * _GC, 8)
              pltpu.make_async_copy(table.at[cxb[b]], rwb[b], gsem[b]).wait()
              pltpu.async_copy(rwb[b], out_hbm.at[pl.ds(2 * (eoff - hpart * _EH), 2 * _GC)],
                               osem[b])
          return 0

      lax.fori_loop(0, _G_FULL // 2, _body, 0)
      for b in range(2):
          pltpu.make_async_copy(
              rwb[b], out_hbm.at[pl.ds(0, 2 * _GC)], osem[b]).wait()

      if _G_TAIL:
          eoff = base + _G_FULL * _GC
          pltpu.sync_copy(ei_hbm.at[0, pl.ds(eoff, _G_TAIL)],
                          ii0.at[pl.ds(0, _G_TAIL)])
          pltpu.sync_copy(ei_hbm.at[1, pl.ds(eoff, _G_TAIL)],
                          ij0.at[pl.ds(0, _G_TAIL)])
          _build_cidx(0, _G_TAIL)
          pltpu.async_copy(
              table.at[cx0.at[pl.ds(0, 2 * _G_TAIL)]],
              rw0.at[pl.ds(0, 2 * _G_TAIL)], gs0).wait()
          pltpu.sync_copy(rw0.at[pl.ds(0, 2 * _G_TAIL)],
                          out_hbm.at[pl.ds(2 * (eoff - hpart * _EH), 2 * _G_TAIL)])

  return _gather_kernel


_gather_half0 = _make_gather(0)
_gather_half1 = _make_gather(1)


_UB = 6400


def _unpack_body(v_ref, o_ref):
    v = v_ref[...]
    r0 = lax.broadcasted_iota(jnp.int32, (128, 1024), 0)
    c0 = lax.broadcasted_iota(jnp.int32, (128, 1024), 1)
    Q = ((r0 // 16 == c0 // 128) & (c0 % 128 == r0 % 16)).astype(jnp.float32)
    u = jnp.dot(v.astype(jnp.bfloat16), Q.astype(jnp.bfloat16),
                preferred_element_type=jnp.float32)
    m = u.reshape(_UB, 128)[:, :16]
    o_ref[...] = m.T


def _unpack_a(out8a):
    v = out8a.reshape(_EH // 8, 128)
    return pl.pallas_call(
        _unpack_body,
        grid=(_EH // _UB,),
        in_specs=[pl.BlockSpec((_UB // 8, 128), lambda i: (i, 0))],
        out_specs=pl.BlockSpec((16, _UB), lambda i: (0, i)),
        out_shape=jax.ShapeDtypeStruct((16, E), jnp.float32),
    )(v)


def _unpack_b(out8b, prev):
    v = out8b.reshape(_EH // 8, 128)

    def body(v_ref, prev_ref, o_ref):
        _unpack_body(v_ref, o_ref)

    nb = _EH // _UB
    outT = pl.pallas_call(
        body,
        grid=(nb,),
        in_specs=[
            pl.BlockSpec((_UB // 8, 128), lambda i: (i, 0)),
            pl.BlockSpec(memory_space=pl.ANY),
        ],
        out_specs=pl.BlockSpec((16, _UB), lambda i: (0, i + nb)),
        out_shape=jax.ShapeDtypeStruct((16, E), jnp.float32),
        input_output_aliases={1: 0},
    )(v, prev)
    return outT.T


def kernel(edge_sh, weight, edge_index):
    mix8 = _mix_call(edge_sh, weight)
    mix = mix8.reshape(E, 16)
    partials = _scatter_kernel(mix, edge_index)
    merged = _add_call(partials)
    m8 = merged.reshape(2 * N_PAD, 8)
    out8a = _gather_half0(m8, edge_index)
    pa = _unpack_a(out8a)
    out8b = _gather_half1(m8, edge_index)
    return _unpack_b(out8b, pa)
